# baseline (speedup 1.0000x reference)
import jax
import jax.numpy as jnp
from jax import lax
from jax.experimental import pallas as pl
from jax.experimental.pallas import tpu as pltpu
from jax.experimental.pallas import tpu_sc as plsc

_B, _T = 4, 8192
_V, _D = 8, 4
_N = _B * _T
_NC, _NS, _L = 2, 16, 16
_NW = _NC * _NS
_IDS_W = _N // _NW
_W_PER_B = _T // _IDS_W
_TBLK = 128
_NBLK_W = _IDS_W // _TBLK
_NT = _T // _TBLK

_mesh = plsc.VectorSubcoreMesh(
    core_axis_name="c", subcore_axis_name="s", num_cores=_NC, num_subcores=_NS
)


@pl.kernel(
    out_type=jax.ShapeDtypeStruct((_B, _NT, _D, _TBLK), jnp.float32),
    mesh=_mesh,
    scratch_types=[
        pltpu.VMEM((_NBLK_W, _TBLK), jnp.int32),
        pltpu.VMEM((_NBLK_W, _D, _TBLK), jnp.float32),
        pltpu.VMEM((_D, _V), jnp.float32),
        pltpu.VMEM((_V * _D,), jnp.float32),
        pltpu.SemaphoreType.DMA,
        pltpu.SemaphoreType.DMA,
    ],
    compiler_params=pltpu.CompilerParams(
        needs_layout_passes=False, use_tc_tiling_on_sc=False
    ),
)
def _embed_sc(ids_hbm, tab_hbm, out_hbm, ids_v, out_v, tab_raw, tab_v, s1, s2):
    wid = lax.axis_index("s") * _NC + lax.axis_index("c")
    b = wid // _W_PER_B
    tb0 = (wid % _W_PER_B) * _NBLK_W
    cp_tab = pltpu.async_copy(tab_hbm, tab_raw, s1)
    cp_ids = pltpu.async_copy(ids_hbm.at[pl.ds(tb0, _NBLK_W), b, :], ids_v, s2)
    cp_tab.wait()

    lanes = lax.iota(jnp.int32, _L)
    row = lanes >> 2
    col = lanes & 3

    tab_v[pl.ds(0, _L)] = plsc.load_gather(tab_raw, [col, row]) + 2.0
    tab_v[pl.ds(_L, _L)] = plsc.load_gather(tab_raw, [col, row + 4]) + 2.0
    cp_ids.wait()

    @plsc.parallel_loop(0, _IDS_W // _L, unroll=2)
    def body(k):
        blk = k >> 3
        base = (k & 7) * _L
        ids4 = ids_v[blk, pl.ds(base, _L)] * 4
        for d in range(_D):
            out_v[blk, d, pl.ds(base, _L)] = plsc.load_gather(tab_v, [ids4 + d])

    pltpu.sync_copy(out_v, out_hbm.at[b, pl.ds(tb0, _NBLK_W), :, :])


def kernel(input_ids, embed):
    ids3 = jnp.transpose(
        input_ids.astype(jnp.int32).reshape(_B, _NT, _TBLK), (1, 0, 2)
    )
    out = _embed_sc(ids3, embed.T)
    return jnp.transpose(out, (0, 1, 3, 2)).reshape(_B, _T, _D)


# speedup vs baseline: 1.0485x; 1.0485x over previous
import jax
import jax.numpy as jnp
from jax import lax
from jax.experimental import pallas as pl
from jax.experimental.pallas import tpu as pltpu
from jax.experimental.pallas import tpu_sc as plsc

_B, _T = 4, 8192
_V, _D = 8, 4
_N = _B * _T
_NC, _NS, _L = 1, 16, 16
_NW = _NC * _NS
_IDS_W = _N // _NW
_W_PER_B = _T // _IDS_W
_TBLK = 128
_NBLK_W = _IDS_W // _TBLK
_NT = _T // _TBLK

_mesh = plsc.VectorSubcoreMesh(
    core_axis_name="c", subcore_axis_name="s", num_cores=_NC, num_subcores=_NS
)


@pl.kernel(
    out_type=jax.ShapeDtypeStruct((_B, _NT, _D, _TBLK), jnp.float32),
    mesh=_mesh,
    scratch_types=[
        pltpu.VMEM((_NBLK_W, _TBLK), jnp.int32),
        pltpu.VMEM((_NBLK_W, _D, _TBLK), jnp.float32),
        pltpu.VMEM((_D, _V), jnp.float32),
        pltpu.VMEM((_V * _D,), jnp.float32),
        pltpu.SemaphoreType.DMA,
        pltpu.SemaphoreType.DMA,
    ],
    compiler_params=pltpu.CompilerParams(
        needs_layout_passes=False, use_tc_tiling_on_sc=False
    ),
)
def _embed_sc(ids_hbm, tab_hbm, out_hbm, ids_v, out_v, tab_raw, tab_v, s1, s2):
    wid = lax.axis_index("s") * _NC + lax.axis_index("c")
    b = wid // _W_PER_B
    tb0 = (wid % _W_PER_B) * _NBLK_W
    cp_tab = pltpu.async_copy(tab_hbm, tab_raw, s1)
    cp_ids = pltpu.async_copy(ids_hbm.at[pl.ds(tb0, _NBLK_W), b, :], ids_v, s2)
    cp_tab.wait()

    lanes = lax.iota(jnp.int32, _L)
    row = lanes >> 2
    col = lanes & 3

    tab_v[pl.ds(0, _L)] = plsc.load_gather(tab_raw, [col, row]) + 2.0
    tab_v[pl.ds(_L, _L)] = plsc.load_gather(tab_raw, [col, row + 4]) + 2.0
    cp_ids.wait()

    @plsc.parallel_loop(0, _IDS_W // _L, unroll=2)
    def body(k):
        blk = k >> 3
        base = (k & 7) * _L
        ids4 = ids_v[blk, pl.ds(base, _L)] * 4
        for d in range(_D):
            out_v[blk, d, pl.ds(base, _L)] = plsc.load_gather(tab_v, [ids4 + d])

    pltpu.sync_copy(out_v, out_hbm.at[b, pl.ds(tb0, _NBLK_W), :, :])


def kernel(input_ids, embed):
    ids3 = jnp.transpose(
        input_ids.astype(jnp.int32).reshape(_B, _NT, _TBLK), (1, 0, 2)
    )
    out = _embed_sc(ids3, embed.T)
    return jnp.transpose(out, (0, 1, 3, 2)).reshape(_B, _T, _D)
